# baseline (speedup 1.0000x reference)
import functools

import jax
import jax.numpy as jnp
from jax import lax
from jax.experimental import pallas as pl
from jax.experimental.pallas import tpu as pltpu
from jax.experimental.pallas import tpu_sc as plsc

VOCAB = 1000000
MAX_LEN = 200
EMBED_DIM = 64
BATCH = 4096

NC = 2
NS = 16
NW = NC * NS

B_PER_W = BATCH // NW
SEQ_PER_CHUNK = 2
CHUNK = SEQ_PER_CHUNK * MAX_LEN
N_CHUNKS = B_PER_W // SEQ_PER_CHUNK
LANES = 16
VPR = EMBED_DIM // LANES
L_STARTS = tuple(range(0, MAX_LEN - LANES, LANES)) + (MAX_LEN - LANES,)


@functools.partial(
    pl.kernel,
    mesh=plsc.VectorSubcoreMesh(core_axis_name="c", subcore_axis_name="s"),
    out_type=jax.ShapeDtypeStruct((BATCH, MAX_LEN, EMBED_DIM), jnp.float32),
    scratch_types=[
        pltpu.VMEM((MAX_LEN, EMBED_DIM), jnp.float32),
        pltpu.VMEM((CHUNK,), jnp.int32),
        pltpu.VMEM((CHUNK,), jnp.int32),
        pltpu.VMEM((CHUNK, 2 * EMBED_DIM), jnp.float32),
        pltpu.VMEM((SEQ_PER_CHUNK, MAX_LEN, EMBED_DIM), jnp.float32),
        pltpu.SemaphoreType.DMA,
    ],
    compiler_params=pltpu.CompilerParams(use_tc_tiling_on_sc=True),
)
def _embed(idx_hbm, t2_hbm, pos_hbm, out_hbm,
           pos_v, idx_v, pair_v, rows_v, out_v, sem):
    wid = lax.axis_index("s") * NC + lax.axis_index("c")
    b_base = wid * B_PER_W
    pltpu.sync_copy(pos_hbm, pos_v)

    def chunk_body(ci, carry):
        bb = b_base + ci * SEQ_PER_CHUNK
        pltpu.sync_copy(idx_hbm.at[pl.ds(bb * MAX_LEN, CHUNK)], idx_v)

        def mk_pairs(v, c):
            sl = pl.ds(v * LANES, LANES)
            pair_v[sl] = lax.shift_right_logical(idx_v[sl], 1)
            return c

        lax.fori_loop(0, CHUNK // LANES, mk_pairs, 0)
        pltpu.async_copy(t2_hbm.at[pair_v], rows_v, sem).wait()

        for s in range(SEQ_PER_CHUNK):
            for l0 in L_STARTS:
                par = idx_v[pl.ds(s * MAX_LEN + l0, LANES)] & 1

                def row_k(k, c, s=s, l0=l0, par=par):
                    ksplat = lax.broadcast_in_dim(k, (LANES,), ())
                    pk = par.at[ksplat].get(mode="promise_in_bounds")
                    pf = pk.astype(jnp.float32)
                    r = s * MAX_LEN + l0 + k
                    for j in range(VPR):
                        lo = rows_v[r, pl.ds(j * LANES, LANES)]
                        hi = rows_v[r, pl.ds(EMBED_DIM + j * LANES, LANES)]
                        out_v[s, l0 + k, pl.ds(j * LANES, LANES)] = (
                            lo + pf * (hi - lo)
                            + pos_v[l0 + k, pl.ds(j * LANES, LANES)]
                        )
                    return c

                lax.fori_loop(0, LANES, row_k, 0)

        pltpu.sync_copy(out_v, out_hbm.at[pl.ds(bb, SEQ_PER_CHUNK)])
        return carry

    lax.fori_loop(0, N_CHUNKS, chunk_body, 0)


def kernel(inputs, token_table, pos_table):
    idx = inputs.reshape(-1).astype(jnp.int32)
    t2 = token_table.reshape(VOCAB // 2, 2 * EMBED_DIM)
    return _embed(idx, t2, pos_table)
